# baseline (speedup 1.0000x reference)
import jax
import jax.numpy as jnp
from jax import lax
from jax.experimental import pallas as pl
from jax.experimental.pallas import tpu as pltpu
from jax.experimental.pallas import tpu_sc as plsc

NC = 2
NS = 16

B = 1024
L = 512
D = 128
BL = B * L
NW = NC * NS
PER_W = BL // NW
K = 128
CHUNKS = PER_W // K
CPS = L // K
NBUF = 4
ROUNDS = CHUNKS // NBUF


def _body(seq_hbm, lab_hbm, tok_hbm, pos_hbm, seg_hbm, out_hbm,
          idx_all, lab_all, bufs, s_sem, a_sem, g_sem, w_sem):
    wid = lax.axis_index("s") * NC + lax.axis_index("c")
    w_base = wid * PER_W

    pltpu.sync_copy(seq_hbm.at[pl.ds(w_base, PER_W)], idx_all)
    pltpu.sync_copy(lab_hbm.at[pl.ds(w_base, PER_W)], lab_all)

    def seed(c_next, slot):
        return pltpu.async_copy(
            pos_hbm.at[pl.ds(slot * K, K)], bufs[slot], s_sem[slot])

    def round_body(r, _):
        for b in range(NBUF):
            c = r * NBUF + b
            buf = bufs[b]
            pltpu.make_async_copy(
                pos_hbm.at[pl.ds(b * K, K)], buf, s_sem[b]).wait()
            idx_sl = idx_all.at[pl.ds(c * K, K)]
            lab_sl = lab_all.at[pl.ds(c * K, K)]
            a_cp = pltpu.async_copy(tok_hbm.at[idx_sl], buf, a_sem[b],
                                    add=True)
            g_cp = pltpu.async_copy(seg_hbm.at[lab_sl], buf, g_sem[b],
                                    add=True)
            nb = (b + 1) % NBUF

            @pl.when(c >= NBUF - 1)
            def _wait_writeout():
                pltpu.make_async_copy(
                    bufs[nb], out_hbm.at[pl.ds(0, K)], w_sem[nb]).wait()

            @pl.when(c + 1 < CHUNKS)
            def _seed_next():
                seed(c + 1, nb)

            a_cp.wait()
            g_cp.wait()
            pltpu.async_copy(buf, out_hbm.at[pl.ds(w_base + c * K, K)],
                             w_sem[b])
        return _

    seed(0, 0)
    lax.fori_loop(0, ROUNDS, round_body, 0)
    for c in range(CHUNKS - NBUF + 1, CHUNKS):
        b = c % NBUF
        pltpu.make_async_copy(
            bufs[b], out_hbm.at[pl.ds(0, K)], w_sem[b]).wait()


@jax.jit
def _embed(seq_flat, lab_flat, token_table, position_table, segment_table):
    mesh = plsc.VectorSubcoreMesh(core_axis_name="c", subcore_axis_name="s")
    kfn = pl.kernel(
        _body,
        out_type=jax.ShapeDtypeStruct((BL, D), jnp.float32),
        mesh=mesh,
        scratch_types=[
            pltpu.VMEM((PER_W,), jnp.int32),
            pltpu.VMEM((PER_W,), jnp.int32),
            [pltpu.VMEM((K, D), jnp.float32) for _ in range(NBUF)],
            [pltpu.SemaphoreType.DMA for _ in range(NBUF)],
            [pltpu.SemaphoreType.DMA for _ in range(NBUF)],
            [pltpu.SemaphoreType.DMA for _ in range(NBUF)],
            [pltpu.SemaphoreType.DMA for _ in range(NBUF)],
        ],
    )
    return kfn(seq_flat, lab_flat, token_table, position_table, segment_table)


def kernel(sequence, segment_label, token_table, position_table, segment_table):
    seq_flat = sequence.reshape(BL).astype(jnp.int32)
    lab_flat = segment_label.reshape(BL).astype(jnp.int32)
    out = _embed(seq_flat, lab_flat, token_table, position_table,
                 segment_table)
    return out.reshape(B, L, D)


# speedup vs baseline: 8.2757x; 8.2757x over previous
import jax
import jax.numpy as jnp
from jax import lax
from jax.experimental import pallas as pl
from jax.experimental.pallas import tpu as pltpu
from jax.experimental.pallas import tpu_sc as plsc

NC = 2
NS = 16
LANES = 16

B = 1024
L = 512
D = 128
BL = B * L
NW = NC * NS
PER_W = BL // NW
K = 128
CHUNKS = PER_W // K
CPS = L // K
NBUF = 4
ROUNDS = CHUNKS // NBUF
GROUPS = D // LANES


def _body(seq_hbm, lab_hbm, tok_hbm, pos_hbm, seg_hbm, out_hbm,
          idx_all, lab_all, seg_v, bufs, s_sem, a_sem, w_sem):
    wid = lax.axis_index("s") * NC + lax.axis_index("c")
    w_base = wid * PER_W

    pltpu.sync_copy(seq_hbm.at[pl.ds(w_base, PER_W)], idx_all)
    pltpu.sync_copy(lab_hbm.at[pl.ds(w_base, PER_W)], lab_all)
    pltpu.sync_copy(seg_hbm, seg_v)

    def seed(slot):
        return pltpu.async_copy(
            pos_hbm.at[pl.ds(slot * K, K)], bufs[slot], s_sem[slot])

    def round_body(r, _):
        for b in range(NBUF):
            c = r * NBUF + b
            buf = bufs[b]
            pltpu.make_async_copy(
                pos_hbm.at[pl.ds(b * K, K)], buf, s_sem[b]).wait()
            idx_sl = idx_all.at[pl.ds(c * K, K)]
            a_cp = pltpu.async_copy(tok_hbm.at[idx_sl], buf, a_sem[b],
                                    add=True)
            nb = (b + 1) % NBUF

            @pl.when(c >= NBUF - 1)
            def _wait_writeout():
                pltpu.make_async_copy(
                    bufs[nb], out_hbm.at[pl.ds(0, K)], w_sem[nb]).wait()

            @pl.when(c + 1 < CHUNKS)
            def _seed_next():
                seed(nb)

            a_cp.wait()
            off = c * K

            def grp_body(t16, carry):
                t0 = t16 * LANES
                lab_vec = lab_all[pl.ds(off + t0, LANES)]
                for j in range(LANES):
                    s = lab_vec[j]

                    @pl.when(s != 0)
                    def _add_seg():
                        for g in range(GROUPS):
                            sl = pl.ds(g * LANES, LANES)
                            buf[t0 + j, sl] = buf[t0 + j, sl] + seg_v[s, sl]

                return carry

            lax.fori_loop(0, K // LANES, grp_body, 0)
            pltpu.async_copy(buf, out_hbm.at[pl.ds(w_base + c * K, K)],
                             w_sem[b])
        return _

    seed(0)
    lax.fori_loop(0, ROUNDS, round_body, 0)
    for c in range(CHUNKS - NBUF + 1, CHUNKS):
        b = c % NBUF
        pltpu.make_async_copy(
            bufs[b], out_hbm.at[pl.ds(0, K)], w_sem[b]).wait()


@jax.jit
def _embed(seq_flat, lab_flat, token_table, position_table, segment_table):
    mesh = plsc.VectorSubcoreMesh(core_axis_name="c", subcore_axis_name="s")
    kfn = pl.kernel(
        _body,
        out_type=jax.ShapeDtypeStruct((BL, D), jnp.float32),
        mesh=mesh,
        scratch_types=[
            pltpu.VMEM((PER_W,), jnp.int32),
            pltpu.VMEM((PER_W,), jnp.int32),
            pltpu.VMEM((3, D), jnp.float32),
            [pltpu.VMEM((K, D), jnp.float32) for _ in range(NBUF)],
            [pltpu.SemaphoreType.DMA for _ in range(NBUF)],
            [pltpu.SemaphoreType.DMA for _ in range(NBUF)],
            [pltpu.SemaphoreType.DMA for _ in range(NBUF)],
        ],
    )
    return kfn(seq_flat, lab_flat, token_table, position_table, segment_table)


def kernel(sequence, segment_label, token_table, position_table, segment_table):
    seq_flat = sequence.reshape(BL).astype(jnp.int32)
    lab_flat = segment_label.reshape(BL).astype(jnp.int32)
    out = _embed(seq_flat, lab_flat, token_table, position_table,
                 segment_table)
    return out.reshape(B, L, D)


# speedup vs baseline: 11.3717x; 1.3741x over previous
import jax
import jax.numpy as jnp
from jax import lax
from jax.experimental import pallas as pl
from jax.experimental.pallas import tpu as pltpu
from jax.experimental.pallas import tpu_sc as plsc

NC = 2
NS = 16
LANES = 16

B = 1024
L = 512
D = 128
BL = B * L
NW = NC * NS
K = 128
NCHUNK = BL // K
CPS = L // K
WPB = NW // CPS
CHUNKS = NCHUNK // NW
NBUF = 4
ROUNDS = CHUNKS // NBUF
GROUPS = D // LANES


def _body(seq_hbm, lab_hbm, tok_hbm, pos_hbm, seg_hbm, out_hbm,
          pos_v, seg_v, idx_ring, lab_ring, bufs, i_sem, a_sem, w_sem):
    wid = lax.axis_index("s") * NC + lax.axis_index("c")
    blk = wid % CPS
    q = wid // CPS

    def chunk_base(k):
        return (blk + CPS * (q * CHUNKS + k)) * K

    pltpu.sync_copy(pos_hbm.at[pl.ds(blk * K, K)], pos_v)
    pltpu.sync_copy(seg_hbm, seg_v)

    def issue_idx(k, slot):
        base = chunk_base(k)
        pltpu.async_copy(seq_hbm.at[pl.ds(base, K)], idx_ring[slot],
                         i_sem[slot])
        pltpu.async_copy(lab_hbm.at[pl.ds(base, K)], lab_ring[slot],
                         i_sem[slot])

    def wait_idx(slot):
        for ring in (idx_ring, lab_ring):
            pltpu.make_async_copy(
                seq_hbm.at[pl.ds(0, K)], ring[slot], i_sem[slot]).wait()

    def issue_gather(slot):
        pltpu.async_copy(tok_hbm.at[idx_ring[slot]], bufs[slot],
                         a_sem[slot], add=False)

    def round_body(r, _):
        for b in range(NBUF):
            k = r * NBUF + b
            buf = bufs[b]
            nb = (b + 1) % NBUF

            @pl.when(k >= NBUF - 1)
            def _wait_writeout():
                pltpu.make_async_copy(
                    bufs[nb], out_hbm.at[pl.ds(0, K)], w_sem[nb]).wait()

            @pl.when(k + 1 < CHUNKS)
            def _issue_next_gather():
                wait_idx(nb)
                issue_gather(nb)

            @pl.when(k + 2 < CHUNKS)
            def _prefetch_indices():
                issue_idx(k + 2, (b + 2) % NBUF)

            pltpu.make_async_copy(
                tok_hbm.at[idx_ring[b]], buf, a_sem[b]).wait()
            labs = lab_ring[b]

            def grp_body(t16, carry):
                t0 = t16 * LANES
                lab_vec = labs[pl.ds(t0, LANES)]
                for j in range(LANES):
                    s = lab_vec[j]
                    for g in range(GROUPS):
                        sl = pl.ds(g * LANES, LANES)
                        buf[t0 + j, sl] = (buf[t0 + j, sl]
                                           + pos_v[t0 + j, sl]
                                           + seg_v[s, sl])
                return carry

            lax.fori_loop(0, K // LANES, grp_body, 0)
            pltpu.async_copy(buf, out_hbm.at[pl.ds(chunk_base(k), K)],
                             w_sem[b])
        return _

    issue_idx(0, 0)
    issue_idx(1, 1)
    wait_idx(0)
    issue_gather(0)
    lax.fori_loop(0, ROUNDS, round_body, 0)
    for k in range(CHUNKS - NBUF + 1, CHUNKS):
        b = k % NBUF
        pltpu.make_async_copy(
            bufs[b], out_hbm.at[pl.ds(0, K)], w_sem[b]).wait()


@jax.jit
def _embed(seq_flat, lab_flat, token_table, position_table, segment_table):
    mesh = plsc.VectorSubcoreMesh(core_axis_name="c", subcore_axis_name="s")
    kfn = pl.kernel(
        _body,
        out_type=jax.ShapeDtypeStruct((BL, D), jnp.float32),
        mesh=mesh,
        scratch_types=[
            pltpu.VMEM((K, D), jnp.float32),
            pltpu.VMEM((3, D), jnp.float32),
            [pltpu.VMEM((K,), jnp.int32) for _ in range(NBUF)],
            [pltpu.VMEM((K,), jnp.int32) for _ in range(NBUF)],
            [pltpu.VMEM((K, D), jnp.float32) for _ in range(NBUF)],
            [pltpu.SemaphoreType.DMA for _ in range(NBUF)],
            [pltpu.SemaphoreType.DMA for _ in range(NBUF)],
            [pltpu.SemaphoreType.DMA for _ in range(NBUF)],
        ],
    )
    return kfn(seq_flat, lab_flat, token_table, position_table, segment_table)


def kernel(sequence, segment_label, token_table, position_table, segment_table):
    seq_flat = sequence.reshape(BL).astype(jnp.int32)
    lab_flat = segment_label.reshape(BL).astype(jnp.int32)
    out = _embed(seq_flat, lab_flat, token_table, position_table,
                 segment_table)
    return out.reshape(B, L, D)


# speedup vs baseline: 12.7944x; 1.1251x over previous
import jax
import jax.numpy as jnp
from jax import lax
from jax.experimental import pallas as pl
from jax.experimental.pallas import tpu as pltpu
from jax.experimental.pallas import tpu_sc as plsc

NC = 2
NS = 16

B = 1024
L = 512
D = 128
BL = B * L
NW = NC * NS
K = 128
CHUNKS = BL // K // NW
NBUF = 4
ROUNDS = CHUNKS // NBUF

RB = 4096
M = BL // RB


def _sc_body(seq_hbm, tok_hbm, out_hbm, idx_ring, bufs, i_sem, a_sem, w_sem):
    wid = lax.axis_index("s") * NC + lax.axis_index("c")
    w_base = wid * CHUNKS * K

    def issue_idx(k, slot):
        pltpu.async_copy(seq_hbm.at[pl.ds(w_base + k * K, K)],
                         idx_ring[slot], i_sem[slot])

    def wait_idx(slot):
        pltpu.make_async_copy(
            seq_hbm.at[pl.ds(0, K)], idx_ring[slot], i_sem[slot]).wait()

    def issue_gather(slot):
        pltpu.async_copy(tok_hbm.at[idx_ring[slot]], bufs[slot],
                         a_sem[slot])

    def round_body(r, _):
        for b in range(NBUF):
            k = r * NBUF + b
            buf = bufs[b]
            nb = (b + 1) % NBUF

            @pl.when(k >= NBUF - 1)
            def _wait_writeout():
                pltpu.make_async_copy(
                    bufs[nb], out_hbm.at[pl.ds(0, K)], w_sem[nb]).wait()

            @pl.when(k + 1 < CHUNKS)
            def _issue_next_gather():
                wait_idx(nb)
                issue_gather(nb)

            @pl.when(k + 2 < CHUNKS)
            def _prefetch_indices():
                issue_idx(k + 2, (b + 2) % NBUF)

            pltpu.make_async_copy(
                tok_hbm.at[idx_ring[b]], buf, a_sem[b]).wait()
            pltpu.async_copy(buf, out_hbm.at[pl.ds(w_base + k * K, K)],
                             w_sem[b])
        return _

    issue_idx(0, 0)
    issue_idx(1, 1)
    wait_idx(0)
    issue_gather(0)
    lax.fori_loop(0, ROUNDS, round_body, 0)
    for k in range(CHUNKS - NBUF + 1, CHUNKS):
        b = k % NBUF
        pltpu.make_async_copy(
            bufs[b], out_hbm.at[pl.ds(0, K)], w_sem[b]).wait()


def _sc_gather(seq_flat, token_table):
    mesh = plsc.VectorSubcoreMesh(core_axis_name="c", subcore_axis_name="s")
    kfn = pl.kernel(
        _sc_body,
        out_type=jax.ShapeDtypeStruct((BL, D), jnp.float32),
        mesh=mesh,
        scratch_types=[
            [pltpu.VMEM((K,), jnp.int32) for _ in range(NBUF)],
            [pltpu.VMEM((K, D), jnp.float32) for _ in range(NBUF)],
            [pltpu.SemaphoreType.DMA for _ in range(NBUF)],
            [pltpu.SemaphoreType.DMA for _ in range(NBUF)],
            [pltpu.SemaphoreType.DMA for _ in range(NBUF)],
        ],
    )
    return kfn(seq_flat, token_table)


def _tc_body(x_ref, lab_ref, pos_ref, seg_ref, out_ref):
    x = x_ref[...]
    lab = lab_ref[...]
    seg1 = seg_ref[1, :]
    seg2 = seg_ref[2, :]
    y = x + pos_ref[...]
    y = (y + jnp.where(lab == 1, seg1[None, :], 0.0)
           + jnp.where(lab == 2, seg2[None, :], 0.0))
    out_ref[...] = y


def _tc_add(gathered, lab2, pos_tiled, segment_table):
    return pl.pallas_call(
        _tc_body,
        grid=(M,),
        in_specs=[
            pl.BlockSpec((RB, D), lambda m: (m, 0)),
            pl.BlockSpec((RB, 1), lambda m: (m, 0)),
            pl.BlockSpec((RB, D), lambda m: (0, 0)),
            pl.BlockSpec((3, D), lambda m: (0, 0)),
        ],
        out_specs=pl.BlockSpec((RB, D), lambda m: (m, 0)),
        out_shape=jax.ShapeDtypeStruct((BL, D), jnp.float32),
    )(gathered, lab2, pos_tiled, segment_table)


@jax.jit
def _embed(seq_flat, lab2, token_table, position_table, segment_table):
    gathered = _sc_gather(seq_flat, token_table)
    pos_tiled = jnp.tile(position_table, (RB // L, 1))
    return _tc_add(gathered, lab2, pos_tiled, segment_table)


def kernel(sequence, segment_label, token_table, position_table, segment_table):
    seq_flat = sequence.reshape(BL).astype(jnp.int32)
    lab2 = segment_label.reshape(BL, 1).astype(jnp.int32)
    out = _embed(seq_flat, lab2, token_table, position_table, segment_table)
    return out.reshape(B, L, D)


# speedup vs baseline: 21.7543x; 1.7003x over previous
import jax
import jax.numpy as jnp
from jax import lax
from jax.experimental import pallas as pl
from jax.experimental.pallas import tpu as pltpu
from jax.experimental.pallas import tpu_sc as plsc

NC = 2
NS = 16
LANES = 16

B = 1024
L = 512
D = 128
BL = B * L
NW = NC * NS
K = 128
CHUNKS = BL // K // NW
CPS = L // K
NBUF = 4
ROUNDS = CHUNKS // NBUF
GROUPS = K // LANES


def _comb_body(pos_ref, seg_ref, out_ref):
    out_ref[...] = seg_ref[...][:, None, :] + pos_ref[...][None, :, :]


def _build_comb(position_table, segment_table):
    comb = pl.pallas_call(
        _comb_body,
        in_specs=[
            pl.BlockSpec((L, D), lambda: (0, 0)),
            pl.BlockSpec((3, D), lambda: (0, 0)),
        ],
        out_specs=pl.BlockSpec((3, L, D), lambda: (0, 0, 0)),
        out_shape=jax.ShapeDtypeStruct((3, L, D), jnp.float32),
    )(position_table, segment_table)
    return comb.reshape(3 * L, D)


def _sc_body(seq_hbm, lab_hbm, tok_hbm, comb_hbm, out_hbm,
             idx_ring, lab_ring, cidx_ring, bufs, i_sem, c_sem, a_sem, w_sem):
    wid = lax.axis_index("s") * NC + lax.axis_index("c")
    w_base = wid * CHUNKS * K
    iota = lax.iota(jnp.int32, LANES)

    def issue_idx(k, slot):
        base = w_base + k * K
        pltpu.async_copy(seq_hbm.at[pl.ds(base, K)], idx_ring[slot],
                         i_sem[slot])
        pltpu.async_copy(lab_hbm.at[pl.ds(base, K)], lab_ring[slot],
                         i_sem[slot])

    def wait_idx(slot):
        for ring in (idx_ring, lab_ring):
            pltpu.make_async_copy(
                seq_hbm.at[pl.ds(0, K)], ring[slot], i_sem[slot]).wait()

    def prep_comb(slot):
        labs = lab_ring[slot]
        cidx = cidx_ring[slot]
        for g in range(GROUPS):
            sl = pl.ds(g * LANES, LANES)
            base = slot * K + g * LANES
            cidx[sl] = labs[sl] * L + (iota + base)
        pltpu.async_copy(comb_hbm.at[cidx], bufs[slot], c_sem[slot])

    def round_body(r, _):
        for b in range(NBUF):
            k = r * NBUF + b
            buf = bufs[b]
            nb = (b + 1) % NBUF

            pltpu.make_async_copy(
                comb_hbm.at[cidx_ring[b]], buf, c_sem[b]).wait()
            a_cp = pltpu.async_copy(tok_hbm.at[idx_ring[b]], buf, a_sem[b],
                                    add=True)

            @pl.when(k + 2 < CHUNKS)
            def _prefetch_indices():
                issue_idx(k + 2, (b + 2) % NBUF)

            @pl.when(k + 1 < CHUNKS)
            def _stage_next():
                wait_idx(nb)

                @pl.when(k >= NBUF - 1)
                def _wait_writeout():
                    pltpu.make_async_copy(
                        bufs[nb], out_hbm.at[pl.ds(0, K)], w_sem[nb]).wait()

                prep_comb(nb)

            a_cp.wait()
            pltpu.async_copy(buf, out_hbm.at[pl.ds(w_base + k * K, K)],
                             w_sem[b])
        return _

    issue_idx(0, 0)
    issue_idx(1, 1)
    wait_idx(0)
    prep_comb(0)
    lax.fori_loop(0, ROUNDS, round_body, 0)
    for k in range(CHUNKS - NBUF + 1, CHUNKS):
        b = k % NBUF
        pltpu.make_async_copy(
            bufs[b], out_hbm.at[pl.ds(0, K)], w_sem[b]).wait()


def _sc_embed(seq_flat, lab_flat, token_table, comb):
    mesh = plsc.VectorSubcoreMesh(core_axis_name="c", subcore_axis_name="s")
    kfn = pl.kernel(
        _sc_body,
        out_type=jax.ShapeDtypeStruct((BL, D), jnp.float32),
        mesh=mesh,
        scratch_types=[
            [pltpu.VMEM((K,), jnp.int32) for _ in range(NBUF)],
            [pltpu.VMEM((K,), jnp.int32) for _ in range(NBUF)],
            [pltpu.VMEM((K,), jnp.int32) for _ in range(NBUF)],
            [pltpu.VMEM((K, D), jnp.float32) for _ in range(NBUF)],
            [pltpu.SemaphoreType.DMA for _ in range(NBUF)],
            [pltpu.SemaphoreType.DMA for _ in range(NBUF)],
            [pltpu.SemaphoreType.DMA for _ in range(NBUF)],
            [pltpu.SemaphoreType.DMA for _ in range(NBUF)],
        ],
    )
    return kfn(seq_flat, lab_flat, token_table, comb)


@jax.jit
def _embed(seq_flat, lab_flat, token_table, position_table, segment_table):
    comb = _build_comb(position_table, segment_table)
    return _sc_embed(seq_flat, lab_flat, token_table, comb)


def kernel(sequence, segment_label, token_table, position_table, segment_table):
    seq_flat = sequence.reshape(BL).astype(jnp.int32)
    lab_flat = segment_label.reshape(BL).astype(jnp.int32)
    out = _embed(seq_flat, lab_flat, token_table, position_table,
                 segment_table)
    return out.reshape(B, L, D)


# speedup vs baseline: 35.6938x; 1.6408x over previous
import jax
import jax.numpy as jnp
from jax import lax
from jax.experimental import pallas as pl
from jax.experimental.pallas import tpu as pltpu
from jax.experimental.pallas import tpu_sc as plsc

NC = 2
NS = 16
LANES = 16

B = 1024
L = 512
D = 128
BL = B * L
NW = NC * NS
K = 128
CHUNKS = BL // K // NW
CPS = L // K
NBUF = 4
ROUNDS = CHUNKS // NBUF
GROUPS = K // LANES


def _comb_body(pos_ref, seg_ref, out_ref):
    out_ref[...] = seg_ref[...][:, None, :] + pos_ref[...][None, :, :]


def _build_comb(position_table, segment_table):
    comb = pl.pallas_call(
        _comb_body,
        in_specs=[
            pl.BlockSpec((L, D), lambda: (0, 0)),
            pl.BlockSpec((3, D), lambda: (0, 0)),
        ],
        out_specs=pl.BlockSpec((3, L, D), lambda: (0, 0, 0)),
        out_shape=jax.ShapeDtypeStruct((3, L, D), jnp.float32),
    )(position_table, segment_table)
    return comb.reshape(3 * L, D)


def _sc_body(seq_hbm, lab_hbm, tok_hbm, comb_hbm, out_hbm,
             comb_sp, idx_ring, lab_ring, cidx_ring, bufs,
             i_sem, c_sem, a_sem, w_sem):
    wid = lax.axis_index("s") * NC + lax.axis_index("c")
    w_base = wid * CHUNKS * K
    iota = lax.iota(jnp.int32, LANES)

    sid = lax.axis_index("s")
    rows = (3 * L) // NS
    pltpu.sync_copy(comb_hbm.at[pl.ds(sid * rows, rows)],
                    comb_sp.at[pl.ds(sid * rows, rows)])
    plsc.subcore_barrier()

    def issue_idx(k, slot):
        base = w_base + k * K
        pltpu.async_copy(seq_hbm.at[pl.ds(base, K)], idx_ring[slot],
                         i_sem[slot])
        pltpu.async_copy(lab_hbm.at[pl.ds(base, K)], lab_ring[slot],
                         i_sem[slot])

    def wait_idx(slot):
        for ring in (idx_ring, lab_ring):
            pltpu.make_async_copy(
                seq_hbm.at[pl.ds(0, K)], ring[slot], i_sem[slot]).wait()

    def prep_comb(k, slot):
        del k
        labs = lab_ring[slot]
        cidx = cidx_ring[slot]
        for g in range(GROUPS):
            sl = pl.ds(g * LANES, LANES)
            base = slot * K + g * LANES
            cidx[sl] = labs[sl] * L + (iota + base)
        pltpu.async_copy(comb_sp.at[cidx], bufs[slot], c_sem[slot])

    def round_body(r, _):
        for b in range(NBUF):
            k = r * NBUF + b
            buf = bufs[b]
            nb = (b + 1) % NBUF
            pb = (b - 1) % NBUF

            pltpu.make_async_copy(
                comb_sp.at[cidx_ring[b]], buf, c_sem[b]).wait()
            pltpu.async_copy(tok_hbm.at[idx_ring[b]], buf, a_sem[b],
                             add=True)

            @pl.when(k + 2 < CHUNKS)
            def _prefetch_indices():
                issue_idx(k + 2, (b + 2) % NBUF)

            @pl.when(k + 1 < CHUNKS)
            def _stage_next():
                wait_idx(nb)

                @pl.when(k >= NBUF - 1)
                def _wait_writeout():
                    pltpu.make_async_copy(
                        bufs[nb], out_hbm.at[pl.ds(0, K)], w_sem[nb]).wait()

                prep_comb(k + 1, nb)

            @pl.when(k >= 1)
            def _retire_prev():
                pltpu.make_async_copy(
                    tok_hbm.at[idx_ring[pb]], bufs[pb], a_sem[pb]).wait()
                pltpu.async_copy(
                    bufs[pb], out_hbm.at[pl.ds(w_base + (k - 1) * K, K)],
                    w_sem[pb])
        return _

    issue_idx(0, 0)
    issue_idx(1, 1)
    wait_idx(0)
    prep_comb(0, 0)
    lax.fori_loop(0, ROUNDS, round_body, 0)
    lb = (CHUNKS - 1) % NBUF
    pltpu.make_async_copy(
        tok_hbm.at[idx_ring[lb]], bufs[lb], a_sem[lb]).wait()
    pltpu.async_copy(bufs[lb],
                     out_hbm.at[pl.ds(w_base + (CHUNKS - 1) * K, K)],
                     w_sem[lb])
    for k in range(CHUNKS - NBUF, CHUNKS):
        b = k % NBUF
        pltpu.make_async_copy(
            bufs[b], out_hbm.at[pl.ds(0, K)], w_sem[b]).wait()


def _sc_embed(seq_flat, lab_flat, token_table, comb):
    mesh = plsc.VectorSubcoreMesh(core_axis_name="c", subcore_axis_name="s")
    kfn = pl.kernel(
        _sc_body,
        out_type=jax.ShapeDtypeStruct((BL, D), jnp.float32),
        mesh=mesh,
        scratch_types=[
            pltpu.VMEM_SHARED((3 * L, D), jnp.float32),
            [pltpu.VMEM((K,), jnp.int32) for _ in range(NBUF)],
            [pltpu.VMEM((K,), jnp.int32) for _ in range(NBUF)],
            [pltpu.VMEM((K,), jnp.int32) for _ in range(NBUF)],
            [pltpu.VMEM((K, D), jnp.float32) for _ in range(NBUF)],
            [pltpu.SemaphoreType.DMA for _ in range(NBUF)],
            [pltpu.SemaphoreType.DMA for _ in range(NBUF)],
            [pltpu.SemaphoreType.DMA for _ in range(NBUF)],
            [pltpu.SemaphoreType.DMA for _ in range(NBUF)],
        ],
    )
    return kfn(seq_flat, lab_flat, token_table, comb)


@jax.jit
def _embed(seq_flat, lab_flat, token_table, position_table, segment_table):
    comb = _build_comb(position_table, segment_table)
    return _sc_embed(seq_flat, lab_flat, token_table, comb)


def kernel(sequence, segment_label, token_table, position_table, segment_table):
    seq_flat = sequence.reshape(BL).astype(jnp.int32)
    lab_flat = segment_label.reshape(BL).astype(jnp.int32)
    out = _embed(seq_flat, lab_flat, token_table, position_table,
                 segment_table)
    return out.reshape(B, L, D)
